# baseline (speedup 1.0000x reference)
import functools

import jax
import jax.numpy as jnp
from jax import lax
from jax.experimental import pallas as pl
from jax.experimental.pallas import tpu as pltpu
from jax.experimental.pallas import tpu_sc as plsc

B, L, V, D = 4096, 200, 1000000, 64
EPS = 1e-6
N = B * L
NC, NS = 2, 16
NW = NC * NS
PER_W = N // NW
T = 256
HW = PER_W // 2
NBLK = HW // T
NJ = T // 128


def _lane_sum_splat(x, perms):
    dnums = lax.GatherDimensionNumbers(
        offset_dims=(), collapsed_slice_dims=(0,), start_index_map=(0,)
    )
    for idx in perms:
        x = x + lax.gather(
            x,
            idx[:, None],
            dnums,
            slice_sizes=(1,),
            mode=lax.GatherScatterMode.PROMISE_IN_BOUNDS,
        )
    return x


def _comb_body(pos_ref, seg_ref, out_ref):
    out_ref[...] = seg_ref[...][:, None, :] + pos_ref[...][None, :, :]


def _build_comb(pos_used, seg_table):
    return pl.pallas_call(
        _comb_body,
        out_shape=jax.ShapeDtypeStruct((3, L, D), jnp.float32),
    )(pos_used, seg_table)


_mesh = plsc.VectorSubcoreMesh(
    core_axis_name="c", subcore_axis_name="s", num_cores=NC, num_subcores=NS
)


@functools.partial(
    pl.kernel,
    out_type=jax.ShapeDtypeStruct((N, D), jnp.float32),
    mesh=_mesh,
    scratch_types=[
        pltpu.VMEM((HW,), jnp.int32),
        pltpu.VMEM((HW,), jnp.int32),
        pltpu.VMEM((T, D), jnp.float32),
        pltpu.VMEM((T, D), jnp.float32),
        pltpu.VMEM((T, D), jnp.float32),
        pltpu.VMEM((T, D), jnp.float32),
        pltpu.VMEM((D,), jnp.float32),
        pltpu.VMEM((D,), jnp.float32),
        pltpu.SemaphoreType.DMA,
        pltpu.SemaphoreType.DMA,
        pltpu.SemaphoreType.DMA,
        pltpu.SemaphoreType.DMA,
    ],
    compiler_params=pltpu.CompilerParams(use_tc_tiling_on_sc=False),
)
def _sc_embed(tok_hbm, seq_hbm, seg_hbm, comb_hbm, gam_hbm, bet_hbm, out_hbm,
              idx_big, crow_big, rows0, rows1, cb0, cb1,
              gam_v, bet_v, semg0, semg1, semw0, semw1):
    wid = lax.axis_index("s") * NC + lax.axis_index("c")
    wbase = wid * PER_W
    rows_s = (rows0, rows1)
    cb_s = (cb0, cb1)
    semg = (semg0, semg1)
    semw = (semw0, semw1)
    pltpu.sync_copy(gam_hbm, gam_v)
    pltpu.sync_copy(bet_hbm, bet_v)
    gam = [gam_v[pl.ds(k * 16, 16)] for k in range(4)]
    bet = [bet_v[pl.ds(k * 16, 16)] for k in range(4)]
    perms = [lax.iota(jnp.int32, 16) ^ off for off in (8, 4, 2, 1)]
    dummy = out_hbm.at[pl.ds(0, T)]

    def prefetch(b, s):
        for j in range(NJ):
            sl = pl.ds(j * 128, 128)
            bsl = pl.ds(b * T + j * 128, 128)
            pltpu.async_copy(
                comb_hbm.at[crow_big.at[bsl]], cb_s[s].at[sl], semg[s]
            )
            pltpu.async_copy(
                tok_hbm.at[idx_big.at[bsl]], rows_s[s].at[sl], semg[s]
            )

    def compute(b, s):
        rows, cb = rows_s[s], cb_s[s]

        @plsc.parallel_loop(0, T, step=1, unroll=6)
        def tok_body(t):
            x = [
                rows[t, pl.ds(k * 16, 16)] + cb[t, pl.ds(k * 16, 16)]
                for k in range(4)
            ]
            s_ = (x[0] + x[1]) + (x[2] + x[3])
            q = (x[0] * x[0] + x[1] * x[1]) + (x[2] * x[2] + x[3] * x[3])
            mean = _lane_sum_splat(s_, perms) * (1.0 / D)
            msq = _lane_sum_splat(q, perms) * (1.0 / D)
            v = (msq - mean * mean) + EPS
            ii = lax.bitcast_convert_type(v, jnp.int32)
            ii = 0x5F3759DF - lax.shift_right_logical(ii, 1)
            y = lax.bitcast_convert_type(ii, jnp.float32)
            h = v * 0.5
            for _ in range(2):
                y = y * (1.5 - h * y * y)
            for k in range(4):
                rows[t, pl.ds(k * 16, 16)] = (x[k] - mean) * y * gam[k] + bet[k]

    def chunk_body(c, carry0):
        cbase = wbase + c * HW
        pltpu.sync_copy(seq_hbm.at[pl.ds(cbase, HW)], idx_big)
        pltpu.sync_copy(seg_hbm.at[pl.ds(cbase, HW)], crow_big)

        @plsc.parallel_loop(0, HW // 16, step=1, unroll=2)
        def crow_grp(g):
            base_t = g * 16
            segv = crow_big[pl.ds(base_t, 16)]
            posv = lax.rem(
                jnp.full((16,), cbase + base_t, jnp.int32)
                + lax.iota(jnp.int32, 16),
                L,
            )
            crow_big[pl.ds(base_t, 16)] = segv * L + posv

        prefetch(0, 0)

        def pair_body(j2, carry):
            for s in (0, 1):
                b = j2 * 2 + s
                s2 = 1 - s

                @pl.when(b + 1 < NBLK)
                def _():
                    @pl.when(b >= 1)
                    def _():
                        pltpu.make_async_copy(
                            rows_s[s2], dummy, semw[s2]
                        ).wait()

                    prefetch(b + 1, s2)

                pltpu.make_async_copy(dummy, rows_s[s], semg[s]).wait()
                pltpu.make_async_copy(dummy, cb_s[s], semg[s]).wait()
                compute(b, s)
                pltpu.async_copy(
                    rows_s[s], out_hbm.at[pl.ds(cbase + b * T, T)], semw[s]
                )
            return carry

        lax.fori_loop(0, NBLK // 2, pair_body, 0)
        pltpu.make_async_copy(rows_s[0], dummy, semw[0]).wait()
        pltpu.make_async_copy(rows_s[1], dummy, semw[1]).wait()
        return carry0

    lax.fori_loop(0, 2, chunk_body, 0)


def kernel(seq, segment_label, token_table, pos_table, seg_table, gamma, beta):
    comb = _build_comb(pos_table[:L], seg_table).reshape(3 * L, D)
    seqf = seq.astype(jnp.int32).reshape(N)
    segf = segment_label.astype(jnp.int32).reshape(N)
    out = _sc_embed(token_table, seqf, segf, comb, gamma, beta)
    return out.reshape(B, L, D)


# speedup vs baseline: 1.0005x; 1.0005x over previous
import functools

import jax
import jax.numpy as jnp
from jax import lax
from jax.experimental import pallas as pl
from jax.experimental.pallas import tpu as pltpu
from jax.experimental.pallas import tpu_sc as plsc

B, L, V, D = 4096, 200, 1000000, 64
EPS = 1e-6
N = B * L
NC, NS = 2, 16
NW = NC * NS
PER_W = N // NW
T = 256
HW = PER_W // 2
NBLK = HW // T
NJ = T // 128


def _lane_sum_splat(x, perms):
    dnums = lax.GatherDimensionNumbers(
        offset_dims=(), collapsed_slice_dims=(0,), start_index_map=(0,)
    )
    for idx in perms:
        x = x + lax.gather(
            x,
            idx[:, None],
            dnums,
            slice_sizes=(1,),
            mode=lax.GatherScatterMode.PROMISE_IN_BOUNDS,
        )
    return x


def _comb_body(pos_ref, seg_ref, out_ref):
    out_ref[...] = seg_ref[...][:, None, :] + pos_ref[...][None, :, :]


def _build_comb(pos_used, seg_table):
    return pl.pallas_call(
        _comb_body,
        out_shape=jax.ShapeDtypeStruct((3, L, D), jnp.float32),
    )(pos_used, seg_table)


_mesh = plsc.VectorSubcoreMesh(
    core_axis_name="c", subcore_axis_name="s", num_cores=NC, num_subcores=NS
)


@functools.partial(
    pl.kernel,
    out_type=jax.ShapeDtypeStruct((N, D), jnp.float32),
    mesh=_mesh,
    scratch_types=[
        pltpu.VMEM((HW,), jnp.int32),
        pltpu.VMEM((HW,), jnp.int32),
        pltpu.VMEM((T, D), jnp.float32),
        pltpu.VMEM((T, D), jnp.float32),
        pltpu.VMEM((T, D), jnp.float32),
        pltpu.VMEM((T, D), jnp.float32),
        pltpu.VMEM((D,), jnp.float32),
        pltpu.VMEM((D,), jnp.float32),
        pltpu.SemaphoreType.DMA,
        pltpu.SemaphoreType.DMA,
        pltpu.SemaphoreType.DMA,
        pltpu.SemaphoreType.DMA,
    ],
    compiler_params=pltpu.CompilerParams(use_tc_tiling_on_sc=False),
)
def _sc_embed(tok_hbm, seq_hbm, seg_hbm, comb_hbm, gam_hbm, bet_hbm, out_hbm,
              idx_big, crow_big, rows0, rows1, cb0, cb1,
              gam_v, bet_v, semg0, semg1, semw0, semw1):
    wid = lax.axis_index("s") * NC + lax.axis_index("c")
    wbase = wid * PER_W
    rows_s = (rows0, rows1)
    cb_s = (cb0, cb1)
    semg = (semg0, semg1)
    semw = (semw0, semw1)
    pltpu.sync_copy(gam_hbm, gam_v)
    pltpu.sync_copy(bet_hbm, bet_v)
    gam = [gam_v[pl.ds(k * 16, 16)] for k in range(4)]
    bet = [bet_v[pl.ds(k * 16, 16)] for k in range(4)]
    perms = [lax.iota(jnp.int32, 16) ^ off for off in (8, 4, 2, 1)]
    dummy = out_hbm.at[pl.ds(0, T)]

    def prefetch(b, s):
        for j in range(NJ):
            sl = pl.ds(j * 128, 128)
            bsl = pl.ds(b * T + j * 128, 128)
            pltpu.async_copy(
                comb_hbm.at[crow_big.at[bsl]], cb_s[s].at[sl], semg[s]
            )
            pltpu.async_copy(
                tok_hbm.at[idx_big.at[bsl]], rows_s[s].at[sl], semg[s]
            )

    def compute(b, s):
        rows, cb = rows_s[s], cb_s[s]

        @plsc.parallel_loop(0, T, step=1, unroll=4)
        def tok_body(t):
            x = [
                rows[t, pl.ds(k * 16, 16)] + cb[t, pl.ds(k * 16, 16)]
                for k in range(4)
            ]
            s_ = (x[0] + x[1]) + (x[2] + x[3])
            q = (x[0] * x[0] + x[1] * x[1]) + (x[2] * x[2] + x[3] * x[3])
            mean = _lane_sum_splat(s_, perms) * (1.0 / D)
            msq = _lane_sum_splat(q, perms) * (1.0 / D)
            v = (msq - mean * mean) + EPS
            ii = lax.bitcast_convert_type(v, jnp.int32)
            ii = 0x5F3759DF - lax.shift_right_logical(ii, 1)
            y = lax.bitcast_convert_type(ii, jnp.float32)
            h = v * 0.5
            for _ in range(2):
                y = y * (1.5 - h * y * y)
            for k in range(4):
                rows[t, pl.ds(k * 16, 16)] = (x[k] - mean) * y * gam[k] + bet[k]

    def chunk_body(c, carry0):
        cbase = wbase + c * HW
        pltpu.sync_copy(seq_hbm.at[pl.ds(cbase, HW)], idx_big)
        pltpu.sync_copy(seg_hbm.at[pl.ds(cbase, HW)], crow_big)

        @plsc.parallel_loop(0, HW // 16, step=1, unroll=2)
        def crow_grp(g):
            base_t = g * 16
            segv = crow_big[pl.ds(base_t, 16)]
            posv = lax.rem(
                jnp.full((16,), cbase + base_t, jnp.int32)
                + lax.iota(jnp.int32, 16),
                L,
            )
            crow_big[pl.ds(base_t, 16)] = segv * L + posv

        prefetch(0, 0)

        def pair_body(j2, carry):
            for s in (0, 1):
                b = j2 * 2 + s
                s2 = 1 - s

                @pl.when(b + 1 < NBLK)
                def _():
                    @pl.when(b >= 1)
                    def _():
                        pltpu.make_async_copy(
                            rows_s[s2], dummy, semw[s2]
                        ).wait()

                    prefetch(b + 1, s2)

                pltpu.make_async_copy(dummy, rows_s[s], semg[s]).wait()
                pltpu.make_async_copy(dummy, cb_s[s], semg[s]).wait()
                compute(b, s)
                pltpu.async_copy(
                    rows_s[s], out_hbm.at[pl.ds(cbase + b * T, T)], semw[s]
                )
            return carry

        lax.fori_loop(0, NBLK // 2, pair_body, 0)
        pltpu.make_async_copy(rows_s[0], dummy, semw[0]).wait()
        pltpu.make_async_copy(rows_s[1], dummy, semw[1]).wait()
        return carry0

    lax.fori_loop(0, 2, chunk_body, 0)


def kernel(seq, segment_label, token_table, pos_table, seg_table, gamma, beta):
    comb = _build_comb(pos_table[:L], seg_table).reshape(3 * L, D)
    seqf = seq.astype(jnp.int32).reshape(N)
    segf = segment_label.astype(jnp.int32).reshape(N)
    out = _sc_embed(token_table, seqf, segf, comb, gamma, beta)
    return out.reshape(B, L, D)


# speedup vs baseline: 1.0026x; 1.0021x over previous
import functools

import jax
import jax.numpy as jnp
from jax import lax
from jax.experimental import pallas as pl
from jax.experimental.pallas import tpu as pltpu
from jax.experimental.pallas import tpu_sc as plsc

B, L, V, D = 4096, 200, 1000000, 64
EPS = 1e-6
N = B * L
NC, NS = 2, 16
NW = NC * NS
PER_W = N // NW
T = 256
HW = PER_W
NBLK = HW // T
NJ = T // 128


def _lane_sum_splat(x, perms):
    dnums = lax.GatherDimensionNumbers(
        offset_dims=(), collapsed_slice_dims=(0,), start_index_map=(0,)
    )
    for idx in perms:
        x = x + lax.gather(
            x,
            idx[:, None],
            dnums,
            slice_sizes=(1,),
            mode=lax.GatherScatterMode.PROMISE_IN_BOUNDS,
        )
    return x


def _comb_body(pos_ref, seg_ref, out_ref):
    out_ref[...] = seg_ref[...][:, None, :] + pos_ref[...][None, :, :]


def _build_comb(pos_used, seg_table):
    return pl.pallas_call(
        _comb_body,
        out_shape=jax.ShapeDtypeStruct((3, L, D), jnp.float32),
    )(pos_used, seg_table)


_mesh = plsc.VectorSubcoreMesh(
    core_axis_name="c", subcore_axis_name="s", num_cores=NC, num_subcores=NS
)


@functools.partial(
    pl.kernel,
    out_type=jax.ShapeDtypeStruct((N, D), jnp.float32),
    mesh=_mesh,
    scratch_types=[
        pltpu.VMEM((HW,), jnp.int32),
        pltpu.VMEM((HW,), jnp.int32),
        pltpu.VMEM((T, D), jnp.float32),
        pltpu.VMEM((T, D), jnp.float32),
        pltpu.VMEM((T, D), jnp.float32),
        pltpu.VMEM((T, D), jnp.float32),
        pltpu.VMEM((D,), jnp.float32),
        pltpu.VMEM((D,), jnp.float32),
        pltpu.SemaphoreType.DMA,
        pltpu.SemaphoreType.DMA,
        pltpu.SemaphoreType.DMA,
        pltpu.SemaphoreType.DMA,
    ],
    compiler_params=pltpu.CompilerParams(use_tc_tiling_on_sc=False),
)
def _sc_embed(tok_hbm, seq_hbm, seg_hbm, comb_hbm, gam_hbm, bet_hbm, out_hbm,
              idx_big, crow_big, rows0, rows1, cb0, cb1,
              gam_v, bet_v, semg0, semg1, semw0, semw1):
    wid = lax.axis_index("s") * NC + lax.axis_index("c")
    wbase = wid * PER_W
    rows_s = (rows0, rows1)
    cb_s = (cb0, cb1)
    semg = (semg0, semg1)
    semw = (semw0, semw1)
    pltpu.sync_copy(gam_hbm, gam_v)
    pltpu.sync_copy(bet_hbm, bet_v)
    gam = [gam_v[pl.ds(k * 16, 16)] for k in range(4)]
    bet = [bet_v[pl.ds(k * 16, 16)] for k in range(4)]
    perms = [lax.iota(jnp.int32, 16) ^ off for off in (8, 4, 2, 1)]
    dummy = out_hbm.at[pl.ds(0, T)]

    def prefetch(b, s):
        for j in range(NJ):
            sl = pl.ds(j * 128, 128)
            bsl = pl.ds(b * T + j * 128, 128)
            pltpu.async_copy(
                comb_hbm.at[crow_big.at[bsl]], cb_s[s].at[sl], semg[s]
            )
            pltpu.async_copy(
                tok_hbm.at[idx_big.at[bsl]], rows_s[s].at[sl], semg[s]
            )

    def compute(b, s):
        rows, cb = rows_s[s], cb_s[s]

        @plsc.parallel_loop(0, T, step=1, unroll=4)
        def tok_body(t):
            x = [
                rows[t, pl.ds(k * 16, 16)] + cb[t, pl.ds(k * 16, 16)]
                for k in range(4)
            ]
            s_ = (x[0] + x[1]) + (x[2] + x[3])
            q = (x[0] * x[0] + x[1] * x[1]) + (x[2] * x[2] + x[3] * x[3])
            mean = _lane_sum_splat(s_, perms) * (1.0 / D)
            msq = _lane_sum_splat(q, perms) * (1.0 / D)
            v = (msq - mean * mean) + EPS
            ii = lax.bitcast_convert_type(v, jnp.int32)
            ii = 0x5F3759DF - lax.shift_right_logical(ii, 1)
            y = lax.bitcast_convert_type(ii, jnp.float32)
            h = v * 0.5
            for _ in range(2):
                y = y * (1.5 - h * y * y)
            for k in range(4):
                rows[t, pl.ds(k * 16, 16)] = (x[k] - mean) * y * gam[k] + bet[k]

    def chunk_body(c, carry0):
        cbase = wbase + c * HW
        pltpu.sync_copy(seq_hbm.at[pl.ds(cbase, HW)], idx_big)
        pltpu.sync_copy(seg_hbm.at[pl.ds(cbase, HW)], crow_big)

        @plsc.parallel_loop(0, HW // 16, step=1, unroll=2)
        def crow_grp(g):
            base_t = g * 16
            segv = crow_big[pl.ds(base_t, 16)]
            posv = lax.rem(
                jnp.full((16,), cbase + base_t, jnp.int32)
                + lax.iota(jnp.int32, 16),
                L,
            )
            crow_big[pl.ds(base_t, 16)] = segv * L + posv

        prefetch(0, 0)

        def pair_body(j2, carry):
            for s in (0, 1):
                b = j2 * 2 + s
                s2 = 1 - s

                @pl.when(b + 1 < NBLK)
                def _():
                    @pl.when(b >= 1)
                    def _():
                        pltpu.make_async_copy(
                            rows_s[s2], dummy, semw[s2]
                        ).wait()

                    prefetch(b + 1, s2)

                pltpu.make_async_copy(dummy, rows_s[s], semg[s]).wait()
                pltpu.make_async_copy(dummy, cb_s[s], semg[s]).wait()
                compute(b, s)
                pltpu.async_copy(
                    rows_s[s], out_hbm.at[pl.ds(cbase + b * T, T)], semw[s]
                )
            return carry

        lax.fori_loop(0, NBLK // 2, pair_body, 0)
        pltpu.make_async_copy(rows_s[0], dummy, semw[0]).wait()
        pltpu.make_async_copy(rows_s[1], dummy, semw[1]).wait()
        return carry0

    lax.fori_loop(0, 1, chunk_body, 0)


def kernel(seq, segment_label, token_table, pos_table, seg_table, gamma, beta):
    comb = _build_comb(pos_table[:L], seg_table).reshape(3 * L, D)
    seqf = seq.astype(jnp.int32).reshape(N)
    segf = segment_label.astype(jnp.int32).reshape(N)
    out = _sc_embed(token_table, seqf, segf, comb, gamma, beta)
    return out.reshape(B, L, D)
